# baseline (speedup 1.0000x reference)
import dataclasses
import functools

import jax
import jax.numpy as jnp
from jax import lax
from jax.experimental import pallas as pl
from jax.experimental.pallas import tpu as pltpu
from jax.experimental.pallas import tpu_sc as plsc

NC = 2
NS = 16
L = 16
NW = NC * NS
B = 16384
D = 32
BPW = B // NW
GCH = 128

_mesh = plsc.VectorSubcoreMesh(core_axis_name="c", subcore_axis_name="s")

_cp = pltpu.CompilerParams()
if "needs_layout_passes" in pltpu.CompilerParams.__dataclass_fields__:
    _cp = dataclasses.replace(_cp, needs_layout_passes=False)
if "use_tc_tiling_on_sc" in pltpu.CompilerParams.__dataclass_fields__:
    _cp = dataclasses.replace(_cp, use_tc_tiling_on_sc=False)


def _gmf_body(uid_hbm, iid_hbm, utab_hbm, itab_hbm, w_hbm, b_hbm, out_hbm,
              uidx_v, iidx_v, urows_v, irows_v, w_v, b_v, o_v, sem):
    wid = lax.axis_index("s") * NC + lax.axis_index("c")
    base = wid * BPW

    pltpu.sync_copy(uid_hbm.at[pl.ds(base, BPW)], uidx_v)
    pltpu.sync_copy(iid_hbm.at[pl.ds(base, BPW)], iidx_v)
    pltpu.sync_copy(w_hbm, w_v)
    pltpu.sync_copy(b_hbm, b_v)

    copies = []
    for c in range(BPW // GCH):
        sl = pl.ds(c * GCH, GCH)
        copies.append(
            pltpu.async_copy(utab_hbm.at[uidx_v.at[sl]], urows_v.at[sl], sem))
        copies.append(
            pltpu.async_copy(itab_hbm.at[iidx_v.at[sl]], irows_v.at[sl], sem))
    for cp in copies:
        cp.wait()

    @pl.loop(0, BPW)
    def _(r):
        for h in range(D // L):
            sl = (r, pl.ds(h * L, L))
            urows_v.at[sl][...] = (
                urows_v.at[sl][...] * irows_v.at[sl][...]
                * w_v.at[pl.ds(h * L, L)][...])

    @pl.loop(0, BPW, step=L)
    def _(g):
        rows = g + lax.iota(jnp.int32, L)
        acc = b_v[...]
        for d in range(D):
            didx = jnp.full((L,), d, jnp.int32)
            acc = acc + plsc.load_gather(urows_v, [rows, didx])
        o_v.at[pl.ds(g, L)][...] = 1.0 / (1.0 + jnp.exp(-acc))

    pltpu.sync_copy(o_v, out_hbm.at[pl.ds(base, BPW)])


@functools.partial(
    pl.kernel,
    out_type=jax.ShapeDtypeStruct((B,), jnp.float32),
    mesh=_mesh,
    scratch_types=[
        pltpu.VMEM((BPW,), jnp.int32),
        pltpu.VMEM((BPW,), jnp.int32),
        pltpu.VMEM((BPW, D), jnp.float32),
        pltpu.VMEM((BPW, D), jnp.float32),
        pltpu.VMEM((D,), jnp.float32),
        pltpu.VMEM((L,), jnp.float32),
        pltpu.VMEM((BPW,), jnp.float32),
        pltpu.SemaphoreType.DMA,
    ],
    compiler_params=_cp,
)
def _gmf_sc(*args):
    _gmf_body(*args)


@jax.jit
def kernel(user_ids, item_ids, user_table, item_table, W, b):
    uid = user_ids.astype(jnp.int32)
    iid = item_ids.astype(jnp.int32)
    w_flat = W.reshape(D).astype(jnp.float32)
    b16 = jnp.full((L,), b[0], dtype=jnp.float32)
    return _gmf_sc(uid, iid, user_table, item_table, w_flat, b16)
